# baseline (speedup 1.0000x reference)
import functools

import jax
import jax.numpy as jnp
from jax import lax
from jax.experimental import pallas as pl
from jax.experimental.pallas import tpu as pltpu
from jax.experimental.pallas import tpu_sc as plsc

DIM = 32
GROUP = 128 // DIM
NW = 32
LANES = 16
BQ = 256
HIST = 50
NCH = (16384 // NW // BQ) * HIST


@jax.jit
def _sc_gather(ids_flat, table4):
    mesh = plsc.VectorSubcoreMesh(core_axis_name="c", subcore_axis_name="s")

    @functools.partial(
        pl.kernel,
        mesh=mesh,
        out_type=jax.ShapeDtypeStruct((HIST, DIM, 16384), jnp.float32),
        scratch_types=[
            pltpu.VMEM((BQ,), jnp.int32),
            pltpu.VMEM((BQ,), jnp.int32),
            pltpu.VMEM((BQ,), jnp.int32),
            pltpu.VMEM((BQ,), jnp.int32),
            pltpu.VMEM((BQ, 128), jnp.float32),
            pltpu.VMEM((BQ, 128), jnp.float32),
            pltpu.VMEM((DIM, BQ), jnp.float32),
            pltpu.VMEM((DIM, BQ), jnp.float32),
            pltpu.SemaphoreType.DMA,
            pltpu.SemaphoreType.DMA,
            pltpu.SemaphoreType.DMA,
            pltpu.SemaphoreType.DMA,
            pltpu.SemaphoreType.DMA,
            pltpu.SemaphoreType.DMA,
        ],
        compiler_params=pltpu.CompilerParams(
            use_tc_tiling_on_sc=False, needs_layout_passes=False),
    )
    def k(ids_hbm, table_hbm, out_hbm, idx0, idx1, grp0, grp1, row0, row1,
          col0, col1, i0, i1, g0, g1, w0, w1):
        idxs = (idx0, idx1)
        grps = (grp0, grp1)
        rows = (row0, row1)
        colb = (col0, col1)
        isem = (i0, i1)
        gsem = (g0, g1)
        wsem = (w0, w1)
        wid = lax.axis_index("s") * 2 + lax.axis_index("c")
        b0 = wid * (16384 // NW)
        iota = lax.iota(jnp.int32, LANES)
        diag = [
            ((iota + d0) & (LANES - 1)) + LANES * dhi
            for dhi in range(DIM // LANES)
            for d0 in range(LANES)
        ]

        def src_off(c):
            l = c // 2
            return l * 16384 + b0 + (c % 2) * BQ

        def idx_start(c, p):
            pltpu.async_copy(
                ids_hbm.at[pl.ds(src_off(c), BQ)], idxs[p], isem[p])

        def idx_wait(p):
            pltpu.make_async_copy(
                ids_hbm.at[pl.ds(b0, BQ)], idxs[p], isem[p]).wait()

        def grp_compute(p):
            def gbody(j, gcarry):
                v = idxs[p][pl.ds(j * LANES, LANES)]
                grps[p][pl.ds(j * LANES, LANES)] = v >> 2
                return gcarry

            lax.fori_loop(0, BQ // LANES, gbody, 0)

        def gather_start(p):
            pltpu.async_copy(table_hbm.at[grps[p]], rows[p], gsem[p])

        def gather_wait(p):
            pltpu.make_async_copy(
                table_hbm.at[grps[p]], rows[p], gsem[p]).wait()

        def write_start(c, p):
            l = c // 2
            boff = b0 + (c % 2) * BQ
            pltpu.async_copy(
                colb[p], out_hbm.at[l, :, pl.ds(boff, BQ)], wsem[p])

        def write_wait(p):
            pltpu.make_async_copy(
                colb[p], out_hbm.at[0, :, pl.ds(b0, BQ)], wsem[p]).wait()

        def transpose(p):
            def tbody(bb, tcarry):
                ridx = iota + bb * LANES
                sub = (idxs[p][pl.ds(bb * LANES, LANES)] & 3) << 5
                for cidx in diag:
                    v = plsc.load_gather(rows[p], [ridx, sub + cidx])
                    plsc.store_scatter(colb[p], [cidx, ridx], v)
                return tcarry

            lax.fori_loop(0, BQ // LANES, tbody, 0)

        idx_start(0, 0)
        idx_wait(0)
        grp_compute(0)
        gather_start(0)
        idx_start(1, 1)
        for c in (0, 1):
            p, q = c % 2, 1 - c % 2
            idx_wait(q)
            grp_compute(q)
            gather_start(q)
            gather_wait(p)
            transpose(p)
            write_start(c, p)
            idx_start(c + 2, p)

        def pair_body(pr, carry):
            for sub in range(2):
                c = 2 * pr + sub
                p, q = sub, 1 - sub
                idx_wait(q)
                grp_compute(q)
                gather_start(q)
                gather_wait(p)
                write_wait(p)
                transpose(p)
                write_start(c, p)
                idx_start(c + 2, p)
            return carry

        lax.fori_loop(1, NCH // 2 - 1, pair_body, 0)

        idx_wait(1)
        grp_compute(1)
        gather_start(1)
        gather_wait(0)
        write_wait(0)
        transpose(0)
        write_start(NCH - 2, 0)
        gather_wait(1)
        write_wait(1)
        transpose(1)
        write_start(NCH - 1, 1)
        write_wait(0)
        write_wait(1)

    return k(ids_flat, table4)


def kernel(ids, table):
    ids_flat = jnp.transpose(ids).reshape(-1).astype(jnp.int32)
    table4 = table.reshape(250000, 128)
    out_rm = _sc_gather(ids_flat, table4)
    return jnp.transpose(out_rm, (2, 0, 1))
